# Initial kernel scaffold; baseline (speedup 1.0000x reference)
import functools

import jax
import jax.numpy as jnp
from jax import lax
from jax.experimental import pallas as pl
from jax.experimental.pallas import tpu as pltpu
from jax.experimental.pallas import tpu_sc as plsc

N_NODES = 10000
D = 128
WPAD = 144
E = 320000
NC, NS = 2, 16
NW = NC * NS
CHUNK = 128
CPW = -(-E // (NW * CHUNK))
E_PAD = NW * CPW * CHUNK
ROWS_PER_TILE = N_NODES // NS


def _sc_aggregate_body(table, src, dst, out, src_v, dst_v, rows_a, rows_b,
                       acc, sem_a, sem_b):
    cid = lax.axis_index("c")
    sid = lax.axis_index("s")
    wid = sid * NC + cid

    pltpu.sync_copy(src.at[wid], src_v)
    pltpu.sync_copy(dst.at[wid], dst_v)

    def _zero_rows(i, _):
        rows_a[lax.div(i, 9), pl.ds(lax.rem(i, 9) * 16, 16)] = jnp.zeros(
            (16,), jnp.float32)
        return 0
    lax.fori_loop(0, CHUNK * (WPAD // 16), _zero_rows, 0)
    base = sid * ROWS_PER_TILE
    for z in range(-(-ROWS_PER_TILE // CHUNK)):
        sz = min(CHUNK, ROWS_PER_TILE - z * CHUNK)
        pltpu.sync_copy(rows_a.at[pl.ds(0, sz)],
                        acc.at[pl.ds(base + z * CHUNK, sz)])
    plsc.subcore_barrier()

    pltpu.async_copy(table.at[src_v.at[0]], rows_a, sem_a)

    def _step(j, _):
        cur = lax.rem(j, 2)

        def _process(buf, sem, nbuf, nsem):
            pltpu.make_async_copy(table.at[src_v.at[j]], buf, sem).wait()

            @pl.when(j + 1 < CPW)
            def _():
                pltpu.async_copy(table.at[src_v.at[j + 1]], nbuf, nsem)
            pltpu.sync_copy(buf, acc.at[dst_v.at[j]], add=True)

        @pl.when(cur == 0)
        def _():
            _process(rows_a, sem_a, rows_b, sem_b)

        @pl.when(cur == 1)
        def _():
            _process(rows_b, sem_b, rows_a, sem_a)
        return 0

    lax.fori_loop(0, CPW, _step, 0)
    plsc.subcore_barrier()

    pltpu.sync_copy(acc.at[pl.ds(base, ROWS_PER_TILE)],
                    out.at[cid, pl.ds(base, ROWS_PER_TILE)])


def _sc_aggregate(table, src_r, dst_r):
    mesh = plsc.VectorSubcoreMesh(core_axis_name="c", subcore_axis_name="s")
    return pl.kernel(
        _sc_aggregate_body,
        out_type=jax.ShapeDtypeStruct((NC, N_NODES, WPAD), jnp.float32),
        mesh=mesh,
        scratch_types=[
            pltpu.VMEM((CPW, CHUNK), jnp.int32),
            pltpu.VMEM((CPW, CHUNK), jnp.int32),
            pltpu.VMEM((CHUNK, WPAD), jnp.float32),
            pltpu.VMEM((CHUNK, WPAD), jnp.float32),
            pltpu.VMEM_SHARED((N_NODES, WPAD), jnp.float32),
            pltpu.SemaphoreType.DMA,
            pltpu.SemaphoreType.DMA,
        ],
    )(table, src_r, dst_r)


def _dense_body(apply_relu, p_ref, x_ref, wl_ref, wr_ref, b_ref, o_ref):
    s = p_ref[0] + p_ref[1]
    cnt = s[:, D:D + 1]
    mean = s[:, :D] / jnp.maximum(cnt, 1.0)
    y = (jnp.dot(mean, wl_ref[...], preferred_element_type=jnp.float32)
         + jnp.dot(x_ref[...], wr_ref[...], preferred_element_type=jnp.float32)
         + b_ref[...])
    if apply_relu:
        y = jnp.maximum(y, 0.0)
        o_ref[:, :D] = y
        o_ref[:, D:D + 1] = jnp.ones_like(cnt)
        o_ref[:, D + 1:] = jnp.zeros((y.shape[0], WPAD - D - 1), jnp.float32)
    else:
        o_ref[...] = y


def _dense(partials, x, W_l, W_r, b, apply_relu):
    B = 2000
    grid = (N_NODES // B,)
    out_w = WPAD if apply_relu else D
    return pl.pallas_call(
        functools.partial(_dense_body, apply_relu),
        grid=grid,
        in_specs=[
            pl.BlockSpec((NC, B, WPAD), lambda i: (0, i, 0)),
            pl.BlockSpec((B, D), lambda i: (i, 0)),
            pl.BlockSpec((D, D), lambda i: (0, 0)),
            pl.BlockSpec((D, D), lambda i: (0, 0)),
            pl.BlockSpec((1, D), lambda i: (0, 0)),
        ],
        out_specs=pl.BlockSpec((B, out_w), lambda i: (i, 0)),
        out_shape=jax.ShapeDtypeStruct((N_NODES, out_w), jnp.float32),
    )(partials, x, W1_l if False else W_l, W_r, b)


def kernel(x, edge_index, W1_l, W1_r, b1, W2_l, W2_r, b2):
    src = edge_index[0].astype(jnp.int32)
    dst = edge_index[1].astype(jnp.int32)
    pad = E_PAD - E
    src_r = jnp.concatenate(
        [src, jnp.full((pad,), N_NODES, jnp.int32)]).reshape(NW, CPW, CHUNK)
    dst_r = jnp.concatenate(
        [dst, jnp.zeros((pad,), jnp.int32)]).reshape(NW, CPW, CHUNK)

    xp = (jnp.zeros((N_NODES + 1, WPAD), jnp.float32)
          .at[:N_NODES, :D].set(x)
          .at[:N_NODES, D].set(1.0))

    p1 = _sc_aggregate(xp, src_r, dst_r)
    h = _dense(p1, x, W1_l, W1_r, b1.reshape(1, D), apply_relu=True)

    hp = jnp.concatenate([h, jnp.zeros((1, WPAD), jnp.float32)], axis=0)
    p2 = _sc_aggregate(hp, src_r, dst_r)
    out = _dense(p2, h[:, :D], W2_l, W2_r, b2.reshape(1, D), apply_relu=False)
    return out


# speedup vs baseline: 4.7037x; 4.7037x over previous
import functools

import jax
import jax.numpy as jnp
from jax import lax
from jax.experimental import pallas as pl
from jax.experimental.pallas import tpu as pltpu
from jax.experimental.pallas import tpu_sc as plsc

N_NODES = 10000
D = 128
WPAD = 144
E = 320000
NC, NS = 2, 16
NW = NC * NS
CHUNK = 128
CPW = -(-E // (NW * CHUNK))
E_PAD = NW * CPW * CHUNK
N_ACC = 10240
ROWS_PER_TILE = N_ACC // NS


def _sc_aggregate_body(table, idx, out, ibuf, rows_a, rows_b, acc,
                       sem_ia, sem_ib, sem_ga, sem_gb):
    cid = lax.axis_index("c")
    sid = lax.axis_index("s")
    wid = sid * NC + cid

    def _zero_rows(i, _):
        rows_a[lax.div(i, 9), pl.ds(lax.rem(i, 9) * 16, 16)] = jnp.zeros(
            (16,), jnp.float32)
        return 0
    lax.fori_loop(0, CHUNK * (WPAD // 16), _zero_rows, 0)
    base = sid * ROWS_PER_TILE
    for z in range(ROWS_PER_TILE // CHUNK):
        pltpu.sync_copy(rows_a, acc.at[pl.ds(base + z * CHUNK, CHUNK)])
    plsc.subcore_barrier()

    rows = (rows_a, rows_b)
    sem_i = (sem_ia, sem_ib)
    sem_g = (sem_ga, sem_gb)

    pltpu.sync_copy(idx.at[wid, 0], ibuf.at[0])
    pltpu.async_copy(idx.at[wid, 1], ibuf.at[1], sem_ib)
    pltpu.async_copy(table.at[ibuf.at[0, 0]], rows_a, sem_ga)

    def _step(g, _):
        def do(p):
            q = 1 - p

            @pl.when(g + 1 < CPW)
            def _():
                pltpu.make_async_copy(idx.at[wid, g + 1], ibuf.at[q],
                                      sem_i[q]).wait()
            pltpu.make_async_copy(table.at[ibuf.at[p, 0]], rows[p],
                                  sem_g[p]).wait()

            @pl.when(g + 1 < CPW)
            def _():
                pltpu.async_copy(table.at[ibuf.at[q, 0]], rows[q], sem_g[q])
            pltpu.sync_copy(rows[p], acc.at[ibuf.at[p, 1]], add=True)

            @pl.when(g + 2 < CPW)
            def _():
                pltpu.async_copy(idx.at[wid, g + 2], ibuf.at[p], sem_i[p])

        par = lax.rem(g, 2)

        @pl.when(par == 0)
        def _():
            do(0)

        @pl.when(par == 1)
        def _():
            do(1)
        return 0

    lax.fori_loop(0, CPW, _step, 0)
    plsc.subcore_barrier()

    pltpu.sync_copy(acc.at[pl.ds(base, ROWS_PER_TILE)],
                    out.at[cid, pl.ds(base, ROWS_PER_TILE)])


def _sc_aggregate(table, idx_r):
    mesh = plsc.VectorSubcoreMesh(core_axis_name="c", subcore_axis_name="s")
    return pl.kernel(
        _sc_aggregate_body,
        out_type=jax.ShapeDtypeStruct((NC, N_ACC, WPAD), jnp.float32),
        mesh=mesh,
        compiler_params=pltpu.CompilerParams(use_tc_tiling_on_sc=False),
        scratch_types=[
            pltpu.VMEM((2, 2, CHUNK), jnp.int32),
            pltpu.VMEM((CHUNK, WPAD), jnp.float32),
            pltpu.VMEM((CHUNK, WPAD), jnp.float32),
            pltpu.VMEM_SHARED((N_ACC, WPAD), jnp.float32),
            pltpu.SemaphoreType.DMA,
            pltpu.SemaphoreType.DMA,
            pltpu.SemaphoreType.DMA,
            pltpu.SemaphoreType.DMA,
        ],
    )(table, idx_r)


def _dense_body(apply_relu, p_ref, x_ref, wl_ref, wr_ref, b_ref, o_ref):
    s = p_ref[0] + p_ref[1]
    cnt = s[:, D:D + 1]
    mean = s[:, :D] / jnp.maximum(cnt, 1.0)
    y = (jnp.dot(mean, wl_ref[...], preferred_element_type=jnp.float32)
         + jnp.dot(x_ref[...], wr_ref[...], preferred_element_type=jnp.float32)
         + b_ref[...])
    if apply_relu:
        y = jnp.maximum(y, 0.0)
        o_ref[:, :D] = y
        o_ref[:, D:D + 1] = jnp.ones_like(cnt)
        o_ref[:, D + 1:] = jnp.zeros((y.shape[0], WPAD - D - 1), jnp.float32)
    else:
        o_ref[...] = y


def _dense(partials, x, W_l, W_r, b, apply_relu):
    B = 2000
    grid = (N_NODES // B,)
    out_w = WPAD if apply_relu else D
    return pl.pallas_call(
        functools.partial(_dense_body, apply_relu),
        grid=grid,
        in_specs=[
            pl.BlockSpec((NC, B, WPAD), lambda i: (0, i, 0)),
            pl.BlockSpec((B, D), lambda i: (i, 0)),
            pl.BlockSpec((D, D), lambda i: (0, 0)),
            pl.BlockSpec((D, D), lambda i: (0, 0)),
            pl.BlockSpec((1, D), lambda i: (0, 0)),
        ],
        out_specs=pl.BlockSpec((B, out_w), lambda i: (i, 0)),
        out_shape=jax.ShapeDtypeStruct((N_NODES, out_w), jnp.float32),
    )(partials, x, W_l, W_r, b)


def kernel(x, edge_index, W1_l, W1_r, b1, W2_l, W2_r, b2):
    src = edge_index[0].astype(jnp.int32)
    dst = edge_index[1].astype(jnp.int32)
    pad = E_PAD - E
    src_r = jnp.concatenate(
        [src, jnp.full((pad,), N_NODES, jnp.int32)]).reshape(NW, CPW, CHUNK)
    dst_r = jnp.concatenate(
        [dst, jnp.zeros((pad,), jnp.int32)]).reshape(NW, CPW, CHUNK)
    idx_r = jnp.stack([src_r, dst_r], axis=2)

    xp = (jnp.zeros((N_NODES + 1, WPAD), jnp.float32)
          .at[:N_NODES, :D].set(x)
          .at[:N_NODES, D].set(1.0))

    p1 = _sc_aggregate(xp, idx_r)
    h = _dense(p1, x, W1_l, W1_r, b1.reshape(1, D), apply_relu=True)

    hp = jnp.concatenate([h, jnp.zeros((1, WPAD), jnp.float32)], axis=0)
    p2 = _sc_aggregate(hp, idx_r)
    out = _dense(p2, h[:, :D], W2_l, W2_r, b2.reshape(1, D), apply_relu=False)
    return out


# speedup vs baseline: 8.7971x; 1.8703x over previous
import functools

import jax
import jax.numpy as jnp
from jax import lax
from jax.experimental import pallas as pl
from jax.experimental.pallas import tpu as pltpu
from jax.experimental.pallas import tpu_sc as plsc

N_NODES = 10000
D = 128
WPAD = 144
E = 320000
NC, NS = 2, 16
NW = NC * NS
CHUNK = 128
CPW = -(-E // (NW * CHUNK))
E_PAD = NW * CPW * CHUNK
N_ACC = 10240
ROWS_PER_TILE = N_ACC // NS


def _sc_aggregate_body(table, idx, out, ibuf, rows_a, rows_b, acc,
                       sem_ia, sem_ib, sem_ga, sem_gb):
    cid = lax.axis_index("c")
    sid = lax.axis_index("s")
    wid = sid * NC + cid

    def _zero_rows(i, _):
        rows_a[lax.div(i, 9), pl.ds(lax.rem(i, 9) * 16, 16)] = jnp.zeros(
            (16,), jnp.float32)
        return 0
    lax.fori_loop(0, CHUNK * (WPAD // 16), _zero_rows, 0)
    base = sid * ROWS_PER_TILE
    for z in range(ROWS_PER_TILE // CHUNK):
        pltpu.sync_copy(rows_a, acc.at[pl.ds(base + z * CHUNK, CHUNK)])
    plsc.subcore_barrier()

    rows = (rows_a, rows_b)
    sem_i = (sem_ia, sem_ib)
    sem_g = (sem_ga, sem_gb)

    pltpu.sync_copy(idx.at[wid, 0], ibuf.at[0])
    pltpu.async_copy(idx.at[wid, 1], ibuf.at[1], sem_ib)
    pltpu.async_copy(table.at[ibuf.at[0, 0]], rows_a, sem_ga)

    def _step(g, _):
        def do(p):
            q = 1 - p

            @pl.when(g + 1 < CPW)
            def _():
                pltpu.make_async_copy(idx.at[wid, g + 1], ibuf.at[q],
                                      sem_i[q]).wait()
            pltpu.make_async_copy(table.at[ibuf.at[p, 0]], rows[p],
                                  sem_g[p]).wait()

            @pl.when(g + 1 < CPW)
            def _():
                pltpu.async_copy(table.at[ibuf.at[q, 0]], rows[q], sem_g[q])
            pltpu.sync_copy(rows[p], acc.at[ibuf.at[p, 1]], add=True)

            @pl.when(g + 2 < CPW)
            def _():
                pltpu.async_copy(idx.at[wid, g + 2], ibuf.at[p], sem_i[p])

        par = lax.rem(g, 2)

        @pl.when(par == 0)
        def _():
            do(0)

        @pl.when(par == 1)
        def _():
            do(1)
        return 0

    lax.fori_loop(0, CPW, _step, 0)
    plsc.subcore_barrier()

    pltpu.sync_copy(acc.at[pl.ds(base, ROWS_PER_TILE)],
                    out.at[cid, pl.ds(base, ROWS_PER_TILE)])


def _sc_aggregate(table, idx_r):
    mesh = plsc.VectorSubcoreMesh(core_axis_name="c", subcore_axis_name="s")
    return pl.kernel(
        _sc_aggregate_body,
        out_type=jax.ShapeDtypeStruct((NC, N_ACC, WPAD), jnp.float32),
        mesh=mesh,
        compiler_params=pltpu.CompilerParams(use_tc_tiling_on_sc=False),
        scratch_types=[
            pltpu.VMEM((2, 2, CHUNK), jnp.int32),
            pltpu.VMEM((CHUNK, WPAD), jnp.float32),
            pltpu.VMEM((CHUNK, WPAD), jnp.float32),
            pltpu.VMEM_SHARED((N_ACC, WPAD), jnp.float32),
            pltpu.SemaphoreType.DMA,
            pltpu.SemaphoreType.DMA,
            pltpu.SemaphoreType.DMA,
            pltpu.SemaphoreType.DMA,
        ],
    )(table, idx_r)


def _dense_body(apply_relu, p_ref, x_ref, wl_ref, wr_ref, b_ref, o_ref):
    s = p_ref[0] + p_ref[1]
    cnt = s[:, D:D + 1]
    mean = s[:, :D] / jnp.maximum(cnt, 1.0)
    y = (jnp.dot(mean, wl_ref[...], preferred_element_type=jnp.float32)
         + jnp.dot(x_ref[...], wr_ref[...], preferred_element_type=jnp.float32)
         + b_ref[...])
    if apply_relu:
        y = jnp.maximum(y, 0.0)
        o_ref[:, :D] = y
        o_ref[:, D:D + 1] = jnp.ones_like(cnt)
        o_ref[:, D + 1:] = jnp.zeros((y.shape[0], WPAD - D - 1), jnp.float32)
    else:
        o_ref[...] = y


def _dense(partials, x, W_l, W_r, b, apply_relu):
    B = 2000
    grid = (N_NODES // B,)
    out_w = WPAD if apply_relu else D
    return pl.pallas_call(
        functools.partial(_dense_body, apply_relu),
        grid=grid,
        in_specs=[
            pl.BlockSpec((NC, B, WPAD), lambda i: (0, i, 0)),
            pl.BlockSpec((B, D), lambda i: (i, 0)),
            pl.BlockSpec((D, D), lambda i: (0, 0)),
            pl.BlockSpec((D, D), lambda i: (0, 0)),
            pl.BlockSpec((1, D), lambda i: (0, 0)),
        ],
        out_specs=pl.BlockSpec((B, out_w), lambda i: (i, 0)),
        out_shape=jax.ShapeDtypeStruct((N_NODES, out_w), jnp.float32),
    )(partials, x, W_l, W_r, b)


def kernel(x, edge_index, W1_l, W1_r, b1, W2_l, W2_r, b2):
    src = edge_index[0].astype(jnp.int32)
    dst = edge_index[1].astype(jnp.int32)
    pad = E_PAD - E
    pad_src = (jnp.arange(pad, dtype=jnp.int32) * 79) % N_NODES
    pad_dst = N_NODES + (jnp.arange(pad, dtype=jnp.int32) % (N_ACC - N_NODES))
    src_r = jnp.concatenate([src, pad_src]).reshape(NW, CPW, CHUNK)
    dst_r = jnp.concatenate([dst, pad_dst]).reshape(NW, CPW, CHUNK)
    idx_r = jnp.stack([src_r, dst_r], axis=2)

    xp = (jnp.zeros((N_NODES + 1, WPAD), jnp.float32)
          .at[:N_NODES, :D].set(x)
          .at[:N_NODES, D].set(1.0))

    p1 = _sc_aggregate(xp, idx_r)
    h = _dense(p1, x, W1_l, W1_r, b1.reshape(1, D), apply_relu=True)

    hp = jnp.concatenate([h, jnp.zeros((1, WPAD), jnp.float32)], axis=0)
    p2 = _sc_aggregate(hp, idx_r)
    out = _dense(p2, h[:, :D], W2_l, W2_r, b2.reshape(1, D), apply_relu=False)
    return out


# speedup vs baseline: 11.7884x; 1.3400x over previous
import functools

import jax
import jax.numpy as jnp
from jax import lax
from jax.experimental import pallas as pl
from jax.experimental.pallas import tpu as pltpu
from jax.experimental.pallas import tpu_sc as plsc

N_NODES = 10000
D = 128
E = 320000
NC, NS = 2, 16
NW = NC * NS
CHUNK = 128
CPW = 80
E_PAD = NW * CPW * CHUNK
N_ACC = 10240
ROWS_PER_TILE = N_ACC // NS
CW = 16


def _sc_aggregate_body(with_counts, table, idx, out, out_cnt, ibuf,
                       rows_a, rows_b, acc, cnt, ones_buf,
                       sem_ia, sem_ib, sem_ga, sem_gb):
    cid = lax.axis_index("c")
    sid = lax.axis_index("s")
    wid = sid * NC + cid

    def _zero_rows(i, _):
        rows_a[lax.div(i, 8), pl.ds(lax.rem(i, 8) * 16, 16)] = jnp.zeros(
            (16,), jnp.float32)
        return 0
    lax.fori_loop(0, CHUNK * (D // 16), _zero_rows, 0)
    base = sid * ROWS_PER_TILE
    for z in range(ROWS_PER_TILE // CHUNK):
        pltpu.sync_copy(rows_a, acc.at[pl.ds(base + z * CHUNK, CHUNK)])
    if with_counts:
        for z in range(ROWS_PER_TILE // CHUNK):
            pltpu.sync_copy(rows_a.at[:, pl.ds(0, CW)],
                            cnt.at[pl.ds(base + z * CHUNK, CHUNK)])

        def _fill_ones(i, _):
            ones_buf[i, pl.ds(0, CW)] = jnp.ones((CW,), jnp.float32)
            return 0
        lax.fori_loop(0, CHUNK, _fill_ones, 0)
    plsc.subcore_barrier()

    rows = (rows_a, rows_b)
    sem_i = (sem_ia, sem_ib)
    sem_g = (sem_ga, sem_gb)
    s_sl = pl.ds(0, CHUNK)
    d_sl = pl.ds(CHUNK, CHUNK)

    pltpu.sync_copy(idx.at[wid, 0], ibuf.at[0])
    pltpu.async_copy(idx.at[wid, 1], ibuf.at[1], sem_ib)
    pltpu.async_copy(table.at[ibuf.at[0, s_sl]], rows_a, sem_ga)

    def _step(g, _):
        def do(p):
            q = 1 - p

            @pl.when(g + 1 < CPW)
            def _():
                pltpu.make_async_copy(idx.at[wid, g + 1], ibuf.at[q],
                                      sem_i[q]).wait()
            pltpu.make_async_copy(table.at[ibuf.at[p, s_sl]], rows[p],
                                  sem_g[p]).wait()

            @pl.when(g + 1 < CPW)
            def _():
                pltpu.async_copy(table.at[ibuf.at[q, s_sl]], rows[q], sem_g[q])
            pltpu.sync_copy(rows[p], acc.at[ibuf.at[p, d_sl]], add=True)
            if with_counts:
                pltpu.sync_copy(ones_buf, cnt.at[ibuf.at[p, d_sl]], add=True)

            @pl.when(g + 2 < CPW)
            def _():
                pltpu.async_copy(idx.at[wid, g + 2], ibuf.at[p], sem_i[p])

        par = lax.rem(g, 2)

        @pl.when(par == 0)
        def _():
            do(0)

        @pl.when(par == 1)
        def _():
            do(1)
        return 0

    lax.fori_loop(0, CPW, _step, 0)
    plsc.subcore_barrier()

    pltpu.sync_copy(acc.at[pl.ds(base, ROWS_PER_TILE)],
                    out.at[cid, pl.ds(base, ROWS_PER_TILE)])
    if with_counts:
        pltpu.sync_copy(cnt.at[pl.ds(base, ROWS_PER_TILE)],
                        out_cnt.at[cid, pl.ds(base, ROWS_PER_TILE),
                                   pl.ds(0, CW)])


def _sc_aggregate(table, idx, with_counts):
    mesh = plsc.VectorSubcoreMesh(core_axis_name="c", subcore_axis_name="s")
    return pl.kernel(
        functools.partial(_sc_aggregate_body, with_counts),
        out_type=(jax.ShapeDtypeStruct((NC, N_ACC, D), jnp.float32),
                  jax.ShapeDtypeStruct((NC, N_ACC, D), jnp.float32)),
        mesh=mesh,
        compiler_params=pltpu.CompilerParams(use_tc_tiling_on_sc=False),
        scratch_types=[
            pltpu.VMEM((2, 2 * CHUNK), jnp.int32),
            pltpu.VMEM((CHUNK, D), jnp.float32),
            pltpu.VMEM((CHUNK, D), jnp.float32),
            pltpu.VMEM_SHARED((N_ACC, D), jnp.float32),
            pltpu.VMEM_SHARED((N_ACC, CW), jnp.float32),
            pltpu.VMEM((CHUNK, CW), jnp.float32),
            pltpu.SemaphoreType.DMA,
            pltpu.SemaphoreType.DMA,
            pltpu.SemaphoreType.DMA,
            pltpu.SemaphoreType.DMA,
        ],
    )(table, idx)


def _dense_body(apply_relu, p_ref, c_ref, x_ref, wl_ref, wr_ref, b_ref,
                o_ref):
    s = p_ref[0] + p_ref[1]
    cnt = c_ref[0, :, 0:1] + c_ref[1, :, 0:1]
    mean = s / jnp.maximum(cnt, 1.0)
    y = (jnp.dot(mean, wl_ref[...], preferred_element_type=jnp.float32)
         + jnp.dot(x_ref[...], wr_ref[...], preferred_element_type=jnp.float32)
         + b_ref[...])
    if apply_relu:
        y = jnp.maximum(y, 0.0)
    o_ref[...] = y


def _dense(partials, counts, x, W_l, W_r, b, apply_relu):
    B = 2000
    grid = (N_NODES // B,)
    return pl.pallas_call(
        functools.partial(_dense_body, apply_relu),
        grid=grid,
        in_specs=[
            pl.BlockSpec((NC, B, D), lambda i: (0, i, 0)),
            pl.BlockSpec((NC, B, D), lambda i: (0, i, 0)),
            pl.BlockSpec((B, D), lambda i: (i, 0)),
            pl.BlockSpec((D, D), lambda i: (0, 0)),
            pl.BlockSpec((D, D), lambda i: (0, 0)),
            pl.BlockSpec((1, D), lambda i: (0, 0)),
        ],
        out_specs=pl.BlockSpec((B, D), lambda i: (i, 0)),
        out_shape=jax.ShapeDtypeStruct((N_NODES, D), jnp.float32),
    )(partials, counts, x, W_l, W_r, b)


def kernel(x, edge_index, W1_l, W1_r, b1, W2_l, W2_r, b2):
    src = edge_index[0].astype(jnp.int32)
    dst = edge_index[1].astype(jnp.int32)
    pad = E_PAD - E
    pad_src = (jnp.arange(pad, dtype=jnp.int32) * 79) % N_NODES
    pad_dst = N_NODES + (jnp.arange(pad, dtype=jnp.int32) % (N_ACC - N_NODES))
    src_r = jnp.concatenate([src, pad_src]).reshape(NW, CPW, CHUNK)
    dst_r = jnp.concatenate([dst, pad_dst]).reshape(NW, CPW, CHUNK)
    idx = jnp.concatenate([src_r, dst_r], axis=2)

    p1, c1 = _sc_aggregate(x, idx, with_counts=True)
    h = _dense(p1, c1, x, W1_l, W1_r, b1.reshape(1, D), apply_relu=True)
    p2, _ = _sc_aggregate(h, idx, with_counts=False)
    out = _dense(p2, c1, h, W2_l, W2_r, b2.reshape(1, D), apply_relu=False)
    return out


# speedup vs baseline: 11.9717x; 1.0155x over previous
import functools

import jax
import jax.numpy as jnp
from jax import lax
from jax.experimental import pallas as pl
from jax.experimental.pallas import tpu as pltpu
from jax.experimental.pallas import tpu_sc as plsc

N_NODES = 10000
D = 128
E = 320000
NC, NS = 2, 16
NW = NC * NS
CHUNK = 128
CPW = 80
E_PAD = NW * CPW * CHUNK
N_ACC = 10240
ROWS_PER_TILE = N_ACC // NS
CW = 16


def _sc_aggregate_body(with_counts, table, idx, out, out_cnt, ibuf,
                       rows_0, rows_1, acc, cnt, ones_buf,
                       sem_i0, sem_i1, sem_g0a, sem_g0b, sem_g1a, sem_g1b):
    cid = lax.axis_index("c")
    sid = lax.axis_index("s")
    wid = sid * NC + cid

    rows = (rows_0, rows_1)
    sem_i = (sem_i0, sem_i1)
    sem_g = ((sem_g0a, sem_g0b), (sem_g1a, sem_g1b))

    def _zero_rows(i, _):
        rows_0[lax.div(i, 8), pl.ds(lax.rem(i, 8) * 16, 16)] = jnp.zeros(
            (16,), jnp.float32)
        return 0
    lax.fori_loop(0, CHUNK * (D // 16), _zero_rows, 0)
    base = sid * ROWS_PER_TILE
    for z in range(ROWS_PER_TILE // CHUNK):
        pltpu.sync_copy(rows_0, acc.at[pl.ds(base + z * CHUNK, CHUNK)])
    if with_counts:
        for z in range(ROWS_PER_TILE // CHUNK):
            pltpu.sync_copy(rows_0.at[:, pl.ds(0, CW)],
                            cnt.at[pl.ds(base + z * CHUNK, CHUNK)])

        def _fill_ones(i, _):
            ones_buf[i, pl.ds(0, CW)] = jnp.ones((CW,), jnp.float32)
            return 0
        lax.fori_loop(0, CHUNK, _fill_ones, 0)
    plsc.subcore_barrier()

    HC = CHUNK // 2
    sa_sl = pl.ds(0, HC)
    sb_sl = pl.ds(HC, HC)
    d_sl = pl.ds(CHUNK, CHUNK)
    ha_sl = pl.ds(0, HC)
    hb_sl = pl.ds(HC, HC)

    def _issue_gathers(p):
        pltpu.async_copy(table.at[ibuf.at[p, sa_sl]], rows[p].at[ha_sl],
                         sem_g[p][0])
        pltpu.async_copy(table.at[ibuf.at[p, sb_sl]], rows[p].at[hb_sl],
                         sem_g[p][1])

    def _wait_gathers(p):
        pltpu.make_async_copy(table.at[ibuf.at[p, sa_sl]], rows[p].at[ha_sl],
                              sem_g[p][0]).wait()
        pltpu.make_async_copy(table.at[ibuf.at[p, sb_sl]], rows[p].at[hb_sl],
                              sem_g[p][1]).wait()

    pltpu.sync_copy(idx.at[wid, 0], ibuf.at[0])
    pltpu.async_copy(idx.at[wid, 1], ibuf.at[1], sem_i1)
    _issue_gathers(0)

    def _step(g, _):
        def do(p):
            q = 1 - p

            @pl.when(g + 1 < CPW)
            def _():
                pltpu.make_async_copy(idx.at[wid, g + 1], ibuf.at[q],
                                      sem_i[q]).wait()
            _wait_gathers(p)

            @pl.when(g + 1 < CPW)
            def _():
                _issue_gathers(q)
            pltpu.sync_copy(rows[p], acc.at[ibuf.at[p, d_sl]], add=True)
            if with_counts:
                pltpu.sync_copy(ones_buf, cnt.at[ibuf.at[p, d_sl]], add=True)

            @pl.when(g + 2 < CPW)
            def _():
                pltpu.async_copy(idx.at[wid, g + 2], ibuf.at[p], sem_i[p])

        par = lax.rem(g, 2)
        for br in range(2):
            @pl.when(par == br)
            def _(br=br):
                do(br)
        return 0

    lax.fori_loop(0, CPW, _step, 0)
    plsc.subcore_barrier()

    pltpu.sync_copy(acc.at[pl.ds(base, ROWS_PER_TILE)],
                    out.at[cid, pl.ds(base, ROWS_PER_TILE)])
    if with_counts:
        pltpu.sync_copy(cnt.at[pl.ds(base, ROWS_PER_TILE)],
                        out_cnt.at[cid, pl.ds(base, ROWS_PER_TILE),
                                   pl.ds(0, CW)])


def _sc_aggregate(table, idx, with_counts):
    mesh = plsc.VectorSubcoreMesh(core_axis_name="c", subcore_axis_name="s")
    return pl.kernel(
        functools.partial(_sc_aggregate_body, with_counts),
        out_type=(jax.ShapeDtypeStruct((NC, N_ACC, D), jnp.float32),
                  jax.ShapeDtypeStruct((NC, N_ACC, D), jnp.float32)),
        mesh=mesh,
        compiler_params=pltpu.CompilerParams(use_tc_tiling_on_sc=False),
        scratch_types=[
            pltpu.VMEM((2, 2 * CHUNK), jnp.int32),
            pltpu.VMEM((CHUNK, D), jnp.float32),
            pltpu.VMEM((CHUNK, D), jnp.float32),
            pltpu.VMEM_SHARED((N_ACC, D), jnp.float32),
            pltpu.VMEM_SHARED((N_ACC, CW), jnp.float32),
            pltpu.VMEM((CHUNK, CW), jnp.float32),
        ] + [pltpu.SemaphoreType.DMA] * 6,
    )(table, idx)


def _dense_body(apply_relu, p_ref, c_ref, x_ref, wl_ref, wr_ref, b_ref,
                o_ref):
    s = p_ref[0] + p_ref[1]
    cnt = c_ref[0, :, 0:1] + c_ref[1, :, 0:1]
    mean = s / jnp.maximum(cnt, 1.0)
    y = (jnp.dot(mean, wl_ref[...], preferred_element_type=jnp.float32)
         + jnp.dot(x_ref[...], wr_ref[...], preferred_element_type=jnp.float32)
         + b_ref[...])
    if apply_relu:
        y = jnp.maximum(y, 0.0)
    o_ref[...] = y


def _dense(partials, counts, x, W_l, W_r, b, apply_relu):
    B = 2000
    grid = (N_NODES // B,)
    return pl.pallas_call(
        functools.partial(_dense_body, apply_relu),
        grid=grid,
        in_specs=[
            pl.BlockSpec((NC, B, D), lambda i: (0, i, 0)),
            pl.BlockSpec((NC, B, D), lambda i: (0, i, 0)),
            pl.BlockSpec((B, D), lambda i: (i, 0)),
            pl.BlockSpec((D, D), lambda i: (0, 0)),
            pl.BlockSpec((D, D), lambda i: (0, 0)),
            pl.BlockSpec((1, D), lambda i: (0, 0)),
        ],
        out_specs=pl.BlockSpec((B, D), lambda i: (i, 0)),
        out_shape=jax.ShapeDtypeStruct((N_NODES, D), jnp.float32),
    )(partials, counts, x, W_l, W_r, b)


def kernel(x, edge_index, W1_l, W1_r, b1, W2_l, W2_r, b2):
    src = edge_index[0].astype(jnp.int32)
    dst = edge_index[1].astype(jnp.int32)
    pad = E_PAD - E
    pad_src = (jnp.arange(pad, dtype=jnp.int32) * 79) % N_NODES
    pad_dst = N_NODES + (jnp.arange(pad, dtype=jnp.int32) % (N_ACC - N_NODES))
    src_r = jnp.concatenate([src, pad_src]).reshape(NW, CPW, CHUNK)
    dst_r = jnp.concatenate([dst, pad_dst]).reshape(NW, CPW, CHUNK)
    idx = jnp.concatenate([src_r, dst_r], axis=2)

    p1, c1 = _sc_aggregate(x, idx, with_counts=True)
    h = _dense(p1, c1, x, W1_l, W1_r, b1.reshape(1, D), apply_relu=True)
    p2, _ = _sc_aggregate(h, idx, with_counts=False)
    out = _dense(p2, c1, h, W2_l, W2_r, b2.reshape(1, D), apply_relu=False)
    return out


# speedup vs baseline: 12.1824x; 1.0176x over previous
import functools

import jax
import jax.numpy as jnp
from jax import lax
from jax.experimental import pallas as pl
from jax.experimental.pallas import tpu as pltpu
from jax.experimental.pallas import tpu_sc as plsc

N_NODES = 10000
D = 128
E = 320000
NC, NS = 2, 16
NW = NC * NS
CHUNK = 128
N_CHUNKS = E // CHUNK
BASE_CPW = N_CHUNKS // NW
EXTRA = N_CHUNKS % NW
N_ACC = 10240
ROWS_PER_TILE = N_ACC // NS
CW = 16


def _sc_aggregate_body(with_counts, table, src, dst, out, out_cnt,
                       ibuf_s, ibuf_d, rows_0, rows_1, acc, cnt, ones_buf,
                       sem_is0, sem_is1, sem_id0, sem_id1,
                       sem_g0a, sem_g0b, sem_g1a, sem_g1b):
    cid = lax.axis_index("c")
    sid = lax.axis_index("s")
    wid = sid * NC + cid
    nreal = BASE_CPW + jnp.where(wid < EXTRA, 1, 0)
    start = BASE_CPW * wid + jnp.minimum(wid, EXTRA)

    rows = (rows_0, rows_1)
    sem_is = (sem_is0, sem_is1)
    sem_id = (sem_id0, sem_id1)
    sem_g = ((sem_g0a, sem_g0b), (sem_g1a, sem_g1b))

    def _zero_rows(i, _):
        rows_0[lax.div(i, 8), pl.ds(lax.rem(i, 8) * 16, 16)] = jnp.zeros(
            (16,), jnp.float32)
        return 0
    lax.fori_loop(0, CHUNK * (D // 16), _zero_rows, 0)
    base = sid * ROWS_PER_TILE
    for z in range(ROWS_PER_TILE // CHUNK):
        pltpu.sync_copy(rows_0, acc.at[pl.ds(base + z * CHUNK, CHUNK)])
    if with_counts:
        for z in range(ROWS_PER_TILE // CHUNK):
            pltpu.sync_copy(rows_0.at[:, pl.ds(0, CW)],
                            cnt.at[pl.ds(base + z * CHUNK, CHUNK)])

        def _fill_ones(i, _):
            ones_buf[i, pl.ds(0, CW)] = jnp.ones((CW,), jnp.float32)
            return 0
        lax.fori_loop(0, CHUNK, _fill_ones, 0)
    plsc.subcore_barrier()

    HC = CHUNK // 2
    ha_sl = pl.ds(0, HC)
    hb_sl = pl.ds(HC, HC)

    def _off(g):
        return (start + g) * CHUNK

    def _issue_idx(g, q, sync=False):
        if sync:
            pltpu.sync_copy(src.at[pl.ds(_off(g), CHUNK)], ibuf_s.at[q])
            pltpu.sync_copy(dst.at[pl.ds(_off(g), CHUNK)], ibuf_d.at[q])
        else:
            pltpu.async_copy(src.at[pl.ds(_off(g), CHUNK)], ibuf_s.at[q],
                             sem_is[q])
            pltpu.async_copy(dst.at[pl.ds(_off(g), CHUNK)], ibuf_d.at[q],
                             sem_id[q])

    def _wait_idx(g, q):
        pltpu.make_async_copy(src.at[pl.ds(_off(g), CHUNK)], ibuf_s.at[q],
                              sem_is[q]).wait()
        pltpu.make_async_copy(dst.at[pl.ds(_off(g), CHUNK)], ibuf_d.at[q],
                              sem_id[q]).wait()

    def _issue_gathers(p):
        pltpu.async_copy(table.at[ibuf_s.at[p, ha_sl]], rows[p].at[ha_sl],
                         sem_g[p][0])
        pltpu.async_copy(table.at[ibuf_s.at[p, hb_sl]], rows[p].at[hb_sl],
                         sem_g[p][1])

    def _wait_gathers(p):
        pltpu.make_async_copy(table.at[ibuf_s.at[p, ha_sl]],
                              rows[p].at[ha_sl], sem_g[p][0]).wait()
        pltpu.make_async_copy(table.at[ibuf_s.at[p, hb_sl]],
                              rows[p].at[hb_sl], sem_g[p][1]).wait()

    _issue_idx(0, 0, sync=True)
    _issue_idx(1, 1)
    _issue_gathers(0)

    def _step(g, _):
        def do(p):
            q = 1 - p

            @pl.when(g + 1 < nreal)
            def _():
                _wait_idx(g + 1, q)
            _wait_gathers(p)

            @pl.when(g + 1 < nreal)
            def _():
                _issue_gathers(q)
            pltpu.sync_copy(rows[p], acc.at[ibuf_d.at[p]], add=True)
            if with_counts:
                pltpu.sync_copy(ones_buf, cnt.at[ibuf_d.at[p]], add=True)

            @pl.when(g + 2 < nreal)
            def _():
                _issue_idx(g + 2, p)

        par = lax.rem(g, 2)
        for br in range(2):
            @pl.when(par == br)
            def _(br=br):
                do(br)
        return 0

    lax.fori_loop(0, nreal, _step, 0)
    plsc.subcore_barrier()

    pltpu.sync_copy(acc.at[pl.ds(base, ROWS_PER_TILE)],
                    out.at[cid, pl.ds(base, ROWS_PER_TILE)])
    if with_counts:
        pltpu.sync_copy(cnt.at[pl.ds(base, ROWS_PER_TILE)],
                        out_cnt.at[cid, pl.ds(base, ROWS_PER_TILE),
                                   pl.ds(0, CW)])


def _sc_aggregate(table, src, dst, with_counts):
    mesh = plsc.VectorSubcoreMesh(core_axis_name="c", subcore_axis_name="s")
    return pl.kernel(
        functools.partial(_sc_aggregate_body, with_counts),
        out_type=(jax.ShapeDtypeStruct((NC, N_ACC, D), jnp.float32),
                  jax.ShapeDtypeStruct((NC, N_ACC, D), jnp.float32)),
        mesh=mesh,
        compiler_params=pltpu.CompilerParams(use_tc_tiling_on_sc=False),
        scratch_types=[
            pltpu.VMEM((2, CHUNK), jnp.int32),
            pltpu.VMEM((2, CHUNK), jnp.int32),
            pltpu.VMEM((CHUNK, D), jnp.float32),
            pltpu.VMEM((CHUNK, D), jnp.float32),
            pltpu.VMEM_SHARED((N_ACC, D), jnp.float32),
            pltpu.VMEM_SHARED((N_ACC, CW), jnp.float32),
            pltpu.VMEM((CHUNK, CW), jnp.float32),
        ] + [pltpu.SemaphoreType.DMA] * 8,
    )(table, src, dst)


def _dense_body(apply_relu, p_ref, c_ref, x_ref, wl_ref, wr_ref, b_ref,
                o_ref):
    s = p_ref[0] + p_ref[1]
    cnt = c_ref[0, :, 0:1] + c_ref[1, :, 0:1]
    mean = s / jnp.maximum(cnt, 1.0)
    y = (jnp.dot(mean, wl_ref[...], preferred_element_type=jnp.float32)
         + jnp.dot(x_ref[...], wr_ref[...], preferred_element_type=jnp.float32)
         + b_ref[...])
    if apply_relu:
        y = jnp.maximum(y, 0.0)
    o_ref[...] = y


def _dense(partials, counts, x, W_l, W_r, b, apply_relu):
    B = 2000
    grid = (N_NODES // B,)
    return pl.pallas_call(
        functools.partial(_dense_body, apply_relu),
        grid=grid,
        in_specs=[
            pl.BlockSpec((NC, B, D), lambda i: (0, i, 0)),
            pl.BlockSpec((NC, B, D), lambda i: (0, i, 0)),
            pl.BlockSpec((B, D), lambda i: (i, 0)),
            pl.BlockSpec((D, D), lambda i: (0, 0)),
            pl.BlockSpec((D, D), lambda i: (0, 0)),
            pl.BlockSpec((1, D), lambda i: (0, 0)),
        ],
        out_specs=pl.BlockSpec((B, D), lambda i: (i, 0)),
        out_shape=jax.ShapeDtypeStruct((N_NODES, D), jnp.float32),
    )(partials, counts, x, W_l, W_r, b)


def kernel(x, edge_index, W1_l, W1_r, b1, W2_l, W2_r, b2):
    src = edge_index[0].astype(jnp.int32)
    dst = edge_index[1].astype(jnp.int32)

    p1, c1 = _sc_aggregate(x, src, dst, with_counts=True)
    h = _dense(p1, c1, x, W1_l, W1_r, b1.reshape(1, D), apply_relu=True)
    p2, _ = _sc_aggregate(h, src, dst, with_counts=False)
    out = _dense(p2, c1, h, W2_l, W2_r, b2.reshape(1, D), apply_relu=False)
    return out
